# initial kernel scaffold (unmeasured)
import jax
import jax.numpy as jnp
from jax import lax
from jax.experimental import pallas as pl
from jax.experimental.pallas import tpu as pltpu


def kernel(
    x,
):
    def body(*refs):
        pass

    out_shape = jax.ShapeDtypeStruct(..., jnp.float32)
    return pl.pallas_call(body, out_shape=out_shape)(...)


# baseline (device time: 31689 ns/iter reference)
import jax
import jax.numpy as jnp
from jax import lax
from jax.experimental import pallas as pl
from jax.experimental.pallas import tpu as pltpu

N_DEV = 16
K = 8


def _topk(vals, k):
    r, c = vals.shape
    iota = lax.broadcasted_iota(jnp.int32, (r, c), 1)
    outs = []
    cur = vals
    for _ in range(k):
        m = jnp.max(cur, axis=1, keepdims=True)
        idx = jnp.min(jnp.where(cur == m, iota, c), axis=1, keepdims=True)
        outs.append(m)
        cur = jnp.where(iota == idx, -jnp.inf, cur)
    return jnp.concatenate(outs, axis=1)


def kernel(x):
    m, n_per = x.shape

    def body(x_ref, out_ref, cand_ref, send_sems, recv_sems):
        my_pos = lax.axis_index("i")

        barrier_sem = pltpu.get_barrier_semaphore()
        for d in range(1, N_DEV):
            pl.semaphore_signal(
                barrier_sem,
                inc=1,
                device_id=lax.rem(my_pos + d, N_DEV),
                device_id_type=pl.DeviceIdType.LOGICAL,
            )
        pl.semaphore_wait(barrier_sem, N_DEV - 1)

        cand_ref[my_pos] = _topk(x_ref[:, :], K)

        sends = []
        for d in range(1, N_DEV):
            tgt = lax.rem(my_pos + d, N_DEV)
            rdma = pltpu.make_async_remote_copy(
                src_ref=cand_ref.at[my_pos],
                dst_ref=cand_ref.at[my_pos],
                send_sem=send_sems.at[tgt],
                recv_sem=recv_sems.at[my_pos],
                device_id=tgt,
                device_id_type=pl.DeviceIdType.LOGICAL,
            )
            rdma.start()
            sends.append(rdma)

        for d in range(1, N_DEV):
            src = lax.rem(my_pos + d, N_DEV)
            recv = pltpu.make_async_remote_copy(
                src_ref=cand_ref.at[src],
                dst_ref=cand_ref.at[src],
                send_sem=send_sems.at[src],
                recv_sem=recv_sems.at[src],
                device_id=src,
                device_id_type=pl.DeviceIdType.LOGICAL,
            )
            recv.wait_recv()

        for rdma in sends:
            rdma.wait_send()

        all_cand = jnp.concatenate(
            [cand_ref[s] for s in range(N_DEV)], axis=1
        )
        out_ref[:, :] = _topk(all_cand, K)

    return pl.pallas_call(
        body,
        out_shape=jax.ShapeDtypeStruct((m, K), jnp.float32),
        in_specs=[pl.BlockSpec(memory_space=pltpu.VMEM)],
        out_specs=pl.BlockSpec(memory_space=pltpu.VMEM),
        scratch_shapes=[
            pltpu.VMEM((N_DEV, m, K), jnp.float32),
            pltpu.SemaphoreType.DMA((N_DEV,)),
            pltpu.SemaphoreType.DMA((N_DEV,)),
        ],
        compiler_params=pltpu.CompilerParams(collective_id=0),
    )(x)
